# baseline (speedup 1.0000x reference)
import jax
import jax.numpy as jnp
from jax.experimental import pallas as pl

C = 96
H = 512
W = 512
R = 16
NB = H // R
BL = R * W


def _tap_weight(wc2, tap):
    j = jax.lax.broadcasted_iota(jnp.int32, (9 * C, C), 0)
    ci = jax.lax.broadcasted_iota(jnp.int32, (9 * C, C), 1)
    sel = (j == ci * 9 + tap).astype(jnp.bfloat16)
    wk = jax.lax.dot_general(wc2, sel, (((1,), (0,)), ((), ())),
                             preferred_element_type=jnp.float32)
    return wk.astype(jnp.bfloat16)


def _conv_block(x34, wc2, tap0):
    acc = None
    for kh in range(3):
        wk = _tap_weight(wc2, kh * 3 + tap0)
        xk = x34[:, kh * W:kh * W + BL]
        d = jax.lax.dot_general(wk, xk, (((1,), (0,)), ((), ())),
                                preferred_element_type=jnp.float32)
        acc = d if acc is None else acc + d
    return acc


def _kernel(up_ref, x_ref, dn_ref, wc_ref, spa_ref, cm_ref, b_ref, out_ref):
    i = pl.program_id(0)
    up = up_ref[...] * jnp.where(i == 0, 0.0, 1.0)
    dn = dn_ref[...] * jnp.where(i == NB - 1, 0.0, 1.0)
    x34 = jnp.concatenate([up, x_ref[...], dn], axis=1).astype(jnp.bfloat16)
    wc2 = wc_ref[...].astype(jnp.bfloat16)

    col = jax.lax.broadcasted_iota(jnp.int32, (1, BL), 1) % W
    m0 = (col != 0).astype(jnp.float32)
    m1 = (col != W - 1).astype(jnp.float32)

    acc = _conv_block(x34, wc2, 1)
    t0 = _conv_block(x34, wc2, 0)
    z = jnp.zeros((C, 1), dtype=jnp.float32)
    sr = jnp.concatenate([z, t0[:, :-1]], axis=1)
    acc = acc + sr * m0
    t2 = _conv_block(x34, wc2, 2)
    sl = jnp.concatenate([t2[:, 1:], z], axis=1)
    acc = acc + sl * m1

    fea = acc + b_ref[...]
    scale = cm_ref[:, 0:1] * spa_ref[...] + cm_ref[:, 1:2]
    out_ref[...] = jnp.maximum(fea * scale, 0.0)


def kernel(x0, spa_mask, Wc, b, ch_mask):
    u = jax.random.uniform(jax.random.key(1234), ch_mask.shape,
                           minval=1e-8, maxval=1.0 - 1e-8)
    g = -jnp.log(-jnp.log(u))
    cm = jax.nn.softmax((ch_mask + g) / 1.0, axis=-1)

    xf = x0.reshape(C, H * W)
    spa = spa_mask.reshape(1, H * W)
    wc2 = Wc.reshape(C, C * 9)
    cm2 = cm.reshape(C, 2)
    bb = b.reshape(C, 1)

    out = pl.pallas_call(
        _kernel,
        grid=(NB,),
        in_specs=[
            pl.BlockSpec((C, W), lambda i: (0, jnp.maximum(i * R - 1, 0))),
            pl.BlockSpec((C, BL), lambda i: (0, i)),
            pl.BlockSpec((C, W), lambda i: (0, jnp.minimum(i * R + R, H - 1))),
            pl.BlockSpec((C, C * 9), lambda i: (0, 0)),
            pl.BlockSpec((1, BL), lambda i: (0, i)),
            pl.BlockSpec((C, 2), lambda i: (0, 0)),
            pl.BlockSpec((C, 1), lambda i: (0, 0)),
        ],
        out_specs=pl.BlockSpec((C, BL), lambda i: (0, i)),
        out_shape=jax.ShapeDtypeStruct((C, H * W), jnp.float32),
    )(xf, xf, xf, wc2, spa, cm2, bb)

    return (out.reshape(1, C, H, W), cm)


# speedup vs baseline: 1.1424x; 1.1424x over previous
import numpy as np

import jax
import jax.numpy as jnp
from jax.experimental import pallas as pl
from jax.experimental.pallas import tpu as pltpu

C = 96
H = 512
W = 512
R = 16
NB = H // R
BL = R * W


def _conv_block(x34, wt, tap0):
    acc = None
    for kh in range(3):
        tap = kh * 3 + tap0
        wk = wt[:, tap * C:tap * C + C]
        xk = x34[:, kh * W:kh * W + BL]
        d = jax.lax.dot_general(wk, xk, (((1,), (0,)), ((), ())),
                                preferred_element_type=jnp.float32)
        acc = d if acc is None else acc + d
    return acc


def _kernel(up_ref, x_ref, dn_ref, wt_ref, spa_ref, cm_ref, b_ref, out_ref):
    i = pl.program_id(0)
    up = up_ref[...] * jnp.where(i == 0, 0.0, 1.0)
    dn = dn_ref[...] * jnp.where(i == NB - 1, 0.0, 1.0)
    x34 = jnp.concatenate([up, x_ref[...], dn], axis=1).astype(jnp.bfloat16)
    wt = wt_ref[...].astype(jnp.bfloat16)

    col = jax.lax.broadcasted_iota(jnp.int32, (1, BL), 1) % W
    m0 = (col != 0).astype(jnp.float32)
    m1 = (col != W - 1).astype(jnp.float32)

    acc = _conv_block(x34, wt, 1)
    t0 = _conv_block(x34, wt, 0)
    z = jnp.zeros((C, 1), dtype=jnp.float32)
    sr = jnp.concatenate([z, t0[:, :-1]], axis=1)
    acc = acc + sr * m0
    t2 = _conv_block(x34, wt, 2)
    sl = jnp.concatenate([t2[:, 1:], z], axis=1)
    acc = acc + sl * m1

    fea = acc + b_ref[...]
    scale = cm_ref[:, 0:1] * spa_ref[...] + cm_ref[:, 1:2]
    out_ref[...] = jnp.maximum(fea * scale, 0.0)


def kernel(x0, spa_mask, Wc, b, ch_mask):
    u = jax.random.uniform(jax.random.key(1234), ch_mask.shape,
                           minval=1e-8, maxval=1.0 - 1e-8)
    g = -jnp.log(-jnp.log(u))
    cm = jax.nn.softmax((ch_mask + g) / 1.0, axis=-1)

    xf = x0.reshape(C, H * W)
    spa = spa_mask.reshape(1, H * W)
    wc2 = Wc.reshape(C, C * 9)
    cm2 = cm.reshape(C, 2)
    bb = b.reshape(C, 1)

    sel = np.zeros((C * 9, C * 9), dtype=np.float32)
    for tap in range(9):
        for ci in range(C):
            sel[ci * 9 + tap, tap * C + ci] = 1.0
    wt = jnp.dot(wc2, jnp.asarray(sel))

    out = pl.pallas_call(
        _kernel,
        grid=(NB,),
        in_specs=[
            pl.BlockSpec((C, W), lambda i: (0, jnp.maximum(i * R - 1, 0))),
            pl.BlockSpec((C, BL), lambda i: (0, i)),
            pl.BlockSpec((C, W), lambda i: (0, jnp.minimum(i * R + R, H - 1))),
            pl.BlockSpec((C, C * 9), lambda i: (0, 0)),
            pl.BlockSpec((1, BL), lambda i: (0, i)),
            pl.BlockSpec((C, 2), lambda i: (0, 0)),
            pl.BlockSpec((C, 1), lambda i: (0, 0)),
        ],
        out_specs=pl.BlockSpec((C, BL), lambda i: (0, i)),
        out_shape=jax.ShapeDtypeStruct((C, H * W), jnp.float32),
        compiler_params=pltpu.CompilerParams(
            dimension_semantics=("parallel",)),
    )(xf, xf, xf, wt, spa, cm2, bb)

    return (out.reshape(1, C, H, W), cm)


# speedup vs baseline: 1.9725x; 1.7267x over previous
import numpy as np

import jax
import jax.numpy as jnp
from jax.experimental import pallas as pl
from jax.experimental.pallas import tpu as pltpu

C = 96
H = 512
W = 512
R = 16
NB = H // R
NC = 2
NJ = NB // NC
BL = R * W
HL = (R + 2) * W


def _in_copies(x_hbm, xbuf, isem, g, s):
    base = g * R - 1
    return [
        pltpu.make_async_copy(
            x_hbm.at[:, jnp.clip(base + r, 0, H - 1), :],
            xbuf.at[s, :, pl.ds(r * W, W)],
            isem.at[s])
        for r in range(R + 2)
    ]


def _out_copies(out_hbm, obuf, osem, g, s):
    return [
        pltpu.make_async_copy(
            obuf.at[s, :, pl.ds(r * W, W)],
            out_hbm.at[:, g * R + r, :],
            osem.at[s])
        for r in range(R)
    ]


def _conv_block(xbuf, s, wt, tap0):
    acc = None
    for kh in range(3):
        tap = kh * 3 + tap0
        wk = wt[:, tap * C:tap * C + C]
        xk = xbuf[s, :, kh * W:kh * W + BL].astype(jnp.bfloat16)
        d = jax.lax.dot_general(wk, xk, (((1,), (0,)), ((), ())),
                                preferred_element_type=jnp.float32)
        acc = d if acc is None else acc + d
    return acc


def _kernel(x_hbm, wt_ref, spa_ref, cm_ref, b_ref, out_hbm,
            xbuf, obuf, isem, osem):
    j = pl.program_id(1)
    g = pl.program_id(0) * NJ + j
    s = jax.lax.rem(j, 2)
    ns = jax.lax.rem(j + 1, 2)

    @pl.when(j == 0)
    def _():
        for c in _in_copies(x_hbm, xbuf, isem, g, s):
            c.start()

    @pl.when(j + 1 < NJ)
    def _():
        for c in _in_copies(x_hbm, xbuf, isem, g + 1, ns):
            c.start()

    for c in _in_copies(x_hbm, xbuf, isem, g, s):
        c.wait()

    @pl.when(g == 0)
    def _():
        xbuf[s, :, 0:W] = jnp.zeros((C, W), jnp.float32)

    @pl.when(g == NB - 1)
    def _():
        xbuf[s, :, (R + 1) * W:(R + 2) * W] = jnp.zeros((C, W), jnp.float32)

    wt = wt_ref[...].astype(jnp.bfloat16)

    col = jax.lax.broadcasted_iota(jnp.int32, (1, BL), 1) % W
    m0 = (col != 0).astype(jnp.float32)
    m1 = (col != W - 1).astype(jnp.float32)

    acc = _conv_block(xbuf, s, wt, 1)
    t0 = _conv_block(xbuf, s, wt, 0)
    z = jnp.zeros((C, 1), dtype=jnp.float32)
    sr = jnp.concatenate([z, t0[:, :-1]], axis=1)
    acc = acc + sr * m0
    t2 = _conv_block(xbuf, s, wt, 2)
    sl = jnp.concatenate([t2[:, 1:], z], axis=1)
    acc = acc + sl * m1

    fea = acc + b_ref[...]
    scale = cm_ref[:, 0:1] * spa_ref[...] + cm_ref[:, 1:2]
    res = jnp.maximum(fea * scale, 0.0)

    @pl.when(j >= 2)
    def _():
        for c in _out_copies(out_hbm, obuf, osem, g - 2, s):
            c.wait()

    obuf[s, :, :] = res
    for c in _out_copies(out_hbm, obuf, osem, g, s):
        c.start()

    @pl.when(j == NJ - 1)
    def _():
        for c in _out_copies(out_hbm, obuf, osem, g - 1, ns):
            c.wait()
        for c in _out_copies(out_hbm, obuf, osem, g, s):
            c.wait()


def kernel(x0, spa_mask, Wc, b, ch_mask):
    u = jax.random.uniform(jax.random.key(1234), ch_mask.shape,
                           minval=1e-8, maxval=1.0 - 1e-8)
    g = -jnp.log(-jnp.log(u))
    cm = jax.nn.softmax((ch_mask + g) / 1.0, axis=-1)

    x3 = x0.reshape(C, H, W)
    spa = spa_mask.reshape(1, H * W)
    wc2 = Wc.reshape(C, C * 9)
    cm2 = cm.reshape(C, 2)
    bb = b.reshape(C, 1)

    sel = np.zeros((C * 9, C * 9), dtype=np.float32)
    for tap in range(9):
        for ci in range(C):
            sel[ci * 9 + tap, tap * C + ci] = 1.0
    wt = jnp.dot(wc2, jnp.asarray(sel))

    out = pl.pallas_call(
        _kernel,
        grid=(NC, NJ),
        in_specs=[
            pl.BlockSpec(memory_space=pltpu.MemorySpace.HBM),
            pl.BlockSpec((C, C * 9), lambda p, j: (0, 0)),
            pl.BlockSpec((1, BL), lambda p, j: (0, p * NJ + j)),
            pl.BlockSpec((C, 2), lambda p, j: (0, 0)),
            pl.BlockSpec((C, 1), lambda p, j: (0, 0)),
        ],
        out_specs=pl.BlockSpec(memory_space=pltpu.MemorySpace.HBM),
        out_shape=jax.ShapeDtypeStruct((C, H, W), jnp.float32),
        scratch_shapes=[
            pltpu.VMEM((2, C, HL), jnp.float32),
            pltpu.VMEM((2, C, BL), jnp.float32),
            pltpu.SemaphoreType.DMA((2,)),
            pltpu.SemaphoreType.DMA((2,)),
        ],
        compiler_params=pltpu.CompilerParams(
            dimension_semantics=("parallel", "arbitrary")),
    )(x3, wt, spa, cm2, bb)

    return (out.reshape(1, C, H, W), cm)
